# baseline (speedup 1.0000x reference)
import jax
import jax.numpy as jnp
from jax.experimental import pallas as pl
from jax.experimental.pallas import tpu as pltpu

_BATCH = 4096
_D = 64
_K = 8192
_L = 8
_B_BLK = 64


def _prep_kernel(cb_ref, cbt_ref, b2_ref):
    cbt = jnp.swapaxes(cb_ref[0], 0, 1)
    cbt_ref[...] = cbt
    b2_ref[0] = jnp.sum(cbt * cbt, axis=0, keepdims=True)


def _vq_kernel(feat_ref, w_ref, b_ref, cbt_ref, b2_ref, logits_ref,
               qsum_ref):
    x = jnp.dot(feat_ref[...], w_ref[...],
                preferred_element_type=jnp.float32) + b_ref[...]
    res = x
    iota = jax.lax.broadcasted_iota(jnp.int32, (_B_BLK, _K), 1)
    for ll in range(_L):
        cbt = cbt_ref[ll * _D:(ll + 1) * _D, :]
        a2 = jnp.sum(res * res, axis=-1, keepdims=True)
        b2 = b2_ref[ll, :, :]
        ndots = jax.lax.dot_general(res * -2.0, cbt,
                                    (((1,), (0,)), ((), ())),
                                    preferred_element_type=jnp.float32)
        d2 = (a2 + b2) + ndots
        neg = -jnp.sqrt(jnp.maximum(d2, 1e-12))
        logits_ref[:, ll, :] = neg

        maxval = jnp.max(neg, axis=-1, keepdims=True)
        ids = jnp.min(jnp.where(neg == maxval, iota, _K), axis=-1,
                      keepdims=True)

        onehot = (iota == ids).astype(jnp.float32)
        quant = jax.lax.dot_general(onehot, cbt, (((1,), (1,)), ((), ())),
                                    preferred_element_type=jnp.float32,
                                    precision=jax.lax.Precision.HIGHEST)
        res = res - quant
    qsum_ref[...] = x - res


def kernel(features, W_proj, b_proj, codebooks):
    b2d = b_proj.reshape(1, _D)
    cbt, b2 = pl.pallas_call(
        _prep_kernel,
        grid=(_L,),
        in_specs=[pl.BlockSpec((1, _K, _D), lambda l: (l, 0, 0))],
        out_specs=(
            pl.BlockSpec((_D, _K), lambda l: (l, 0)),
            pl.BlockSpec((1, 1, _K), lambda l: (l, 0, 0)),
        ),
        out_shape=(
            jax.ShapeDtypeStruct((_L * _D, _K), jnp.float32),
            jax.ShapeDtypeStruct((_L, 1, _K), jnp.float32),
        ),
    )(codebooks)

    grid = (_BATCH // _B_BLK,)
    logits, qsum = pl.pallas_call(
        _vq_kernel,
        grid=grid,
        in_specs=[
            pl.BlockSpec((_B_BLK, _D), lambda i: (i, 0)),
            pl.BlockSpec((_D, _D), lambda i: (0, 0)),
            pl.BlockSpec((1, _D), lambda i: (0, 0)),
            pl.BlockSpec((_L * _D, _K), lambda i: (0, 0)),
            pl.BlockSpec((_L, 1, _K), lambda i: (0, 0, 0)),
        ],
        out_specs=(
            pl.BlockSpec((_B_BLK, _L, _K), lambda i: (i, 0, 0)),
            pl.BlockSpec((_B_BLK, _D), lambda i: (i, 0)),
        ),
        out_shape=(
            jax.ShapeDtypeStruct((_BATCH, _L, _K), jnp.float32),
            jax.ShapeDtypeStruct((_BATCH, _D), jnp.float32),
        ),
        compiler_params=pltpu.CompilerParams(
            vmem_limit_bytes=100 * 1024 * 1024,
        ),
    )(features, W_proj, b2d, cbt, b2)
    return (logits, qsum)


# speedup vs baseline: 1.1738x; 1.1738x over previous
import jax
import jax.numpy as jnp
from jax.experimental import pallas as pl
from jax.experimental.pallas import tpu as pltpu

_BATCH = 4096
_D = 64
_K = 8192
_L = 8
_B_BLK = 128


def _vq_kernel(feat_ref, w_ref, b_ref, cb_ref, logits_ref, qsum_ref,
               x_ref, res_ref, cbt_ref, b2_ref):
    level = pl.program_id(0)
    i = pl.program_id(1)
    rows = pl.ds(i * _B_BLK, _B_BLK)

    @pl.when(i == 0)
    def _per_level():
        cbt = jnp.swapaxes(cb_ref[0], 0, 1)
        cbt_ref[...] = cbt
        b2_ref[...] = jnp.sum(cbt * cbt, axis=0, keepdims=True)

    @pl.when(level == 0)
    def _init():
        x = jnp.dot(feat_ref[...], w_ref[...],
                    preferred_element_type=jnp.float32) + b_ref[...]
        x_ref[rows, :] = x
        res_ref[rows, :] = x

    res = res_ref[rows, :]
    a2 = jnp.sum(res * res, axis=-1, keepdims=True)
    ndots = jax.lax.dot_general(res * -2.0, cbt_ref[...],
                                (((1,), (0,)), ((), ())),
                                preferred_element_type=jnp.float32)
    d2 = (a2 + b2_ref[...]) + ndots
    neg = -jnp.sqrt(jnp.maximum(d2, 1e-12))
    logits_ref[...] = neg

    maxval = jnp.max(neg, axis=-1, keepdims=True)
    iota = jax.lax.broadcasted_iota(jnp.int32, neg.shape, 1)
    ids = jnp.min(jnp.where(neg == maxval, iota, _K), axis=-1,
                  keepdims=True)

    onehot = (iota == ids).astype(jnp.float32)
    quant = jax.lax.dot_general(onehot, cb_ref[0], (((1,), (0,)), ((), ())),
                                preferred_element_type=jnp.float32,
                                precision=jax.lax.Precision.HIGHEST)
    res_ref[rows, :] = res - quant

    @pl.when(level == _L - 1)
    def _finish():
        qsum_ref[...] = x_ref[rows, :] - res_ref[rows, :]


def kernel(features, W_proj, b_proj, codebooks):
    b2d = b_proj.reshape(1, _D)
    grid = (_L, _BATCH // _B_BLK)
    logits, qsum = pl.pallas_call(
        _vq_kernel,
        grid=grid,
        in_specs=[
            pl.BlockSpec((_B_BLK, _D), lambda l, i: (i, 0)),
            pl.BlockSpec((_D, _D), lambda l, i: (0, 0)),
            pl.BlockSpec((1, _D), lambda l, i: (0, 0)),
            pl.BlockSpec((1, _K, _D), lambda l, i: (l, 0, 0)),
        ],
        out_specs=(
            pl.BlockSpec((_B_BLK, _K), lambda l, i: (i, l)),
            pl.BlockSpec((_B_BLK, _D), lambda l, i: (i, 0)),
        ),
        out_shape=(
            jax.ShapeDtypeStruct((_BATCH, _L * _K), jnp.float32),
            jax.ShapeDtypeStruct((_BATCH, _D), jnp.float32),
        ),
        scratch_shapes=[
            pltpu.VMEM((_BATCH, _D), jnp.float32),
            pltpu.VMEM((_BATCH, _D), jnp.float32),
            pltpu.VMEM((_D, _K), jnp.float32),
            pltpu.VMEM((1, _K), jnp.float32),
        ],
        compiler_params=pltpu.CompilerParams(
            vmem_limit_bytes=100 * 1024 * 1024,
        ),
    )(features, W_proj, b2d, codebooks)
    return (logits.reshape(_BATCH, _L, _K), qsum)
